# baseline (speedup 1.0000x reference)
import jax
import jax.numpy as jnp
from jax.experimental import pallas as pl
from jax.experimental.pallas import tpu as pltpu

_N_CLASSES = 1000
_D1 = 26
_BC = 40


def _onehot_body(idx_ref, out_ref):
    c0 = pl.program_id(0) * _BC
    idx = idx_ref[...]
    iota = c0 + jax.lax.broadcasted_iota(jnp.int32, (_D1, _BC, 4096), 1)
    out_ref[...] = (iota == idx[:, None, :]).astype(jnp.float32)


def kernel(input):
    idx_t = input.T
    grid = _N_CLASSES // _BC
    out = pl.pallas_call(
        _onehot_body,
        grid=(grid,),
        in_specs=[pl.BlockSpec((_D1, 4096), lambda i: (0, 0))],
        out_specs=pl.BlockSpec((_D1, _BC, 4096), lambda i: (0, i, 0)),
        out_shape=jax.ShapeDtypeStruct((_D1, _N_CLASSES, 4096), jnp.float32),
        compiler_params=pltpu.CompilerParams(
            dimension_semantics=("arbitrary",),
        ),
    )(idx_t)
    return jnp.transpose(out, (2, 0, 1))


# speedup vs baseline: 1.0243x; 1.0243x over previous
import jax
import jax.numpy as jnp
from jax.experimental import pallas as pl
from jax.experimental.pallas import tpu as pltpu

_N_CLASSES = 1000
_D1 = 26
_BB = 128


def _onehot_body(idx_ref, out_ref):
    idx = idx_ref[...]
    iota = jax.lax.broadcasted_iota(jnp.int32, (_D1, _N_CLASSES, _BB), 1)
    out_ref[...] = (iota == idx[:, None, :]).astype(jnp.float32)


def kernel(input):
    idx_t = input.T
    grid = 4096 // _BB
    out = pl.pallas_call(
        _onehot_body,
        grid=(grid,),
        in_specs=[pl.BlockSpec((_D1, _BB), lambda i: (0, i))],
        out_specs=pl.BlockSpec((_D1, _N_CLASSES, _BB), lambda i: (0, 0, i)),
        out_shape=jax.ShapeDtypeStruct((_D1, _N_CLASSES, 4096), jnp.float32),
        compiler_params=pltpu.CompilerParams(
            dimension_semantics=("arbitrary",),
        ),
    )(idx_t)
    return jnp.transpose(out, (2, 0, 1))
